# baseline (speedup 1.0000x reference)
import functools

import jax
import jax.numpy as jnp
import numpy as np
from jax import lax
from jax.experimental import pallas as pl
from jax.experimental.pallas import tpu as pltpu
from jax.experimental.pallas import tpu_sc as plsc

KNN = 32
C = 128
B = 8
N = 1024
ROWS = B * N
NC = 2
NS = 16
NW = NC * NS
QPW = ROWS // NW
G = 4
NG = QPW // G
INV_SQRT_C = 1.0 / np.sqrt(C)


def _leaky(x):
    return jnp.where(x >= 0, x, 0.2 * x)


def _proj_body(f_ref, k1w_ref, k1b_ref, v1w_ref, v1b_ref,
               k2w_ref, k2b_ref, v2w_ref, v2b_ref,
               ekv1_ref, ekv2_ref):
    f = f_ref[...]

    def proj(kw, kb, vw, vb, out_ref):
        k = jnp.dot(f, kw[...].T, preferred_element_type=jnp.float32) + kb[...]
        v = jnp.dot(f, vw[...].T, preferred_element_type=jnp.float32) + vb[...]
        out_ref[:, :C] = jnp.exp(-k * INV_SQRT_C)
        out_ref[:, C:] = v

    proj(k1w_ref, k1b_ref, v1w_ref, v1b_ref, ekv1_ref)
    proj(k2w_ref, k2b_ref, v2w_ref, v2b_ref, ekv2_ref)


def _projections(feature2d, k1w, k1b, v1w, v1b, k2w, k2b, v2w, v2b):
    TILE = 1024
    grid = ROWS // TILE
    wspec = pl.BlockSpec((C, C), lambda i: (0, 0))
    bspec = pl.BlockSpec((C,), lambda i: (0,))
    return pl.pallas_call(
        _proj_body,
        grid=(grid,),
        in_specs=[pl.BlockSpec((TILE, C), lambda i: (i, 0)),
                  wspec, bspec, wspec, bspec, wspec, bspec, wspec, bspec],
        out_specs=[pl.BlockSpec((TILE, 2 * C), lambda i: (i, 0)),
                   pl.BlockSpec((TILE, 2 * C), lambda i: (i, 0))],
        out_shape=[jax.ShapeDtypeStruct((ROWS, 2 * C), jnp.float32),
                   jax.ShapeDtypeStruct((ROWS, 2 * C), jnp.float32)],
    )(feature2d, k1w, k1b, v1w, v1b, k2w, k2b, v2w, v2b)


def _attn_body(ekv1_hbm, gidx1_hbm, ekv2_hbm, gidx2_hbm,
               ctx1_hbm, ctx2_hbm,
               idx_v, rows_v, out_v, sem):
    wid = lax.axis_index("s") * NC + lax.axis_index("c")
    qbase = wid * QPW

    for ekv_hbm, gidx_hbm, ctx_hbm in ((ekv1_hbm, gidx1_hbm, ctx1_hbm),
                                       (ekv2_hbm, gidx2_hbm, ctx2_hbm)):
        @pl.loop(0, NG)
        def _group(g):
            base = qbase + g * G
            pltpu.sync_copy(gidx_hbm.at[pl.ds(base * KNN, G * KNN)], idx_v)
            pltpu.async_copy(ekv_hbm.at[idx_v], rows_v, sem).wait()
            for qi in range(G):
                r0 = qi * KNN

                def spass(j, s):
                    r = r0 + j
                    return tuple(s[c] + rows_v[r, pl.ds(c * 16, 16)]
                                 for c in range(8))

                s = lax.fori_loop(
                    0, KNN, spass,
                    tuple(jnp.zeros((16,), jnp.float32) for _ in range(8)))

                def mpass(j, m):
                    r = r0 + j
                    out = []
                    for c in range(8):
                        t = rows_v[r, pl.ds(c * 16, 16)] - s[c]
                        p = t * rows_v[r, pl.ds(C + c * 16, 16)]
                        out.append(jnp.maximum(m[c], p))
                    return tuple(out)

                m = lax.fori_loop(
                    0, KNN, mpass,
                    tuple(jnp.full((16,), -jnp.inf, jnp.float32)
                          for _ in range(8)))

                for c in range(8):
                    out_v[qi, pl.ds(c * 16, 16)] = m[c] / s[c]
            pltpu.sync_copy(out_v, ctx_hbm.at[pl.ds(base, G)])


def _sc_attention(ekv1, gidx1, ekv2, gidx2):
    mesh = plsc.VectorSubcoreMesh(core_axis_name="c", subcore_axis_name="s",
                                  num_cores=NC, num_subcores=NS)
    fn = pl.kernel(
        _attn_body,
        out_type=[jax.ShapeDtypeStruct((ROWS, C), jnp.float32),
                  jax.ShapeDtypeStruct((ROWS, C), jnp.float32)],
        mesh=mesh,
        scratch_types=[
            pltpu.VMEM((G * KNN,), jnp.int32),
            pltpu.VMEM((G * KNN, 2 * C), jnp.float32),
            pltpu.VMEM((G, C), jnp.float32),
            pltpu.SemaphoreType.DMA,
        ],
    )
    return fn(ekv1, gidx1, ekv2, gidx2)


def _merge_tail_body(ctx1_ref, ctx2_ref, res_ref,
                     f1w_ref, f1b_ref, f1g_ref, f1be_ref,
                     f2w_ref, f2b_ref, f2g_ref, f2be_ref,
                     fcw_ref, fcb_ref, fcg_ref, fcbe_ref,
                     out_ref):
    eps = 1e-5

    def lin_bn_act(x, w, b, g, be):
        h = jnp.dot(x, w.T, preferred_element_type=jnp.float32) + b
        mean = jnp.mean(h, axis=0, keepdims=True)
        var = jnp.mean((h - mean) ** 2, axis=0, keepdims=True)
        hn = g * (h - mean) / jnp.sqrt(var + eps) + be
        return _leaky(hn)

    m1 = res_ref[...] + lin_bn_act(ctx1_ref[...], f1w_ref[...], f1b_ref[...],
                                   f1g_ref[...], f1be_ref[...])
    m2 = res_ref[...] + lin_bn_act(ctx2_ref[...], f2w_ref[...], f2b_ref[...],
                                   f2g_ref[...], f2be_ref[...])
    merged = jnp.concatenate([m1, m2], axis=1)
    out_ref[...] = lin_bn_act(merged, fcw_ref[...], fcb_ref[...],
                              fcg_ref[...], fcbe_ref[...])


def _merge_tail(ctx1, ctx2, res, f1w, f1b, f1g, f1be, f2w, f2b, f2g, f2be,
                fcw, fcb, fcg, fcbe):
    out = pl.pallas_call(
        _merge_tail_body,
        out_shape=jax.ShapeDtypeStruct((ROWS, C), jnp.float32),
    )(ctx1, ctx2, res, f1w, f1b, f1g, f1be, f2w, f2b, f2g, f2be,
      fcw, fcb, fcg, fcbe)
    return out.reshape(B, N, C)


def _square_distance(src, dst):
    d = -2.0 * jnp.matmul(src, jnp.swapaxes(dst, 1, 2))
    d = d + jnp.sum(src ** 2, -1)[:, :, None]
    d = d + jnp.sum(dst ** 2, -1)[:, None, :]
    return d


def _knn_point(nsample, xyz, new_xyz):
    sqr = _square_distance(new_xyz, xyz)
    neg_d, idx = jax.lax.top_k(-sqr, nsample)
    return -neg_d, idx


def kernel(xyz, base_xyz, feature, t1_qw, t1_qb, t1_kw, t1_kb, t1_vw, t1_vb, t1_fw, t1_fb, t1_fg, t1_fbe, t2_qw, t2_qb, t2_kw, t2_kb, t2_vw, t2_vb, t2_fw, t2_fb, t2_fg, t2_fbe, fc2_w, fc2_b, fc2_g, fc2_be):
    dist, idx = _knn_point(KNN, base_xyz, xyz)
    _, idx_feature = _knn_point(KNN, feature, feature)

    offs = (jnp.arange(B, dtype=jnp.int32) * N)[:, None, None]
    gidx1 = (idx.astype(jnp.int32) + offs).reshape(-1)
    gidx2 = (idx_feature.astype(jnp.int32) + offs).reshape(-1)

    f2d = feature.reshape(ROWS, C)
    ekv1, ekv2 = _projections(f2d, t1_kw, t1_kb, t1_vw, t1_vb,
                              t2_kw, t2_kb, t2_vw, t2_vb)
    ctx1, ctx2 = _sc_attention(ekv1, gidx1, ekv2, gidx2)

    merge_features = _merge_tail(
        ctx1, ctx2, f2d,
        t1_fw, t1_fb, t1_fg, t1_fbe,
        t2_fw, t2_fb, t2_fg, t2_fbe,
        fc2_w, fc2_b, fc2_g, fc2_be)
    return merge_features, idx, dist
